# baseline (speedup 1.0000x reference)
import functools

import jax
import jax.numpy as jnp
from jax import lax
from jax.experimental import pallas as pl
from jax.experimental.pallas import tpu as pltpu
from jax.experimental.pallas import tpu_sc as plsc

NUM_EMB = 1024
DIM = 64
ROWS = 16 * 576
NCHUNK = 4
BLK = ROWS // NCHUNK
GDIM = 128


def _argmin_body(x_ref, emb_ref, idx_ref, se_ref, iota_ref, emb2_ref):
    e = emb_ref[...]
    se_ref[...] = jnp.sum(e * e, axis=1)[None, :]
    iota_ref[...] = lax.broadcasted_iota(
        jnp.int32, (1, NUM_EMB), 1).astype(jnp.float32)
    emb2_ref[...] = e * (-2.0)

    x = x_ref[...]
    sx = jnp.sum(x * x, axis=1, keepdims=True)
    dot2 = lax.dot_general(x, emb2_ref[...], (((1,), (1,)), ((), ())),
                           preferred_element_type=jnp.float32)
    d = (sx + se_ref[...]) + dot2
    m = jnp.min(d, axis=1, keepdims=True)
    cand = jnp.where(d == m, iota_ref[...], float(NUM_EMB))
    idx_ref[...] = jnp.min(cand, axis=1).astype(jnp.int32).reshape(1, 1, BLK)


def _chunk_indices(flat, emb, c):
    return pl.pallas_call(
        _argmin_body,
        grid=(1,),
        in_specs=[
            pl.BlockSpec((BLK, DIM), lambda i, c=c: (c, 0)),
            pl.BlockSpec((NUM_EMB, DIM), lambda i: (0, 0)),
        ],
        out_specs=pl.BlockSpec((1, 1, BLK), lambda i: (0, 0, 0)),
        out_shape=jax.ShapeDtypeStruct((1, 1, BLK), jnp.int32),
        scratch_shapes=[pltpu.VMEM((1, NUM_EMB), jnp.float32),
                        pltpu.VMEM((1, NUM_EMB), jnp.float32),
                        pltpu.VMEM((NUM_EMB, DIM), jnp.float32)],
    )(flat, emb).reshape(BLK)


@functools.cache
def _make_gather():
    info = plsc.get_sparse_core_info()
    nw = info.num_cores * info.num_subcores
    b_per_w = BLK // nw

    @functools.partial(
        pl.kernel,
        out_type=jax.ShapeDtypeStruct((BLK, GDIM), jnp.float32),
        mesh=plsc.VectorSubcoreMesh(core_axis_name="c", subcore_axis_name="s"),
        scratch_types=[
            pltpu.VMEM((b_per_w,), jnp.int32),
            pltpu.VMEM((b_per_w, GDIM), jnp.float32),
            pltpu.SemaphoreType.DMA,
        ],
    )
    def _gather_rows(emb_hbm, idx_hbm, out_hbm, idx_v, rows_v, sem):
        wid = lax.axis_index("s") * info.num_cores + lax.axis_index("c")
        base = wid * b_per_w
        pltpu.sync_copy(idx_hbm.at[pl.ds(base, b_per_w)], idx_v)
        pltpu.async_copy(emb_hbm.at[idx_v], rows_v, sem).wait()
        pltpu.sync_copy(rows_v, out_hbm.at[pl.ds(base, b_per_w)])

    return _gather_rows


def kernel(inputs, emb_weight):
    b, s, c = inputs.shape
    flat = inputs.reshape(b * s, c)
    emb_pad = jnp.pad(emb_weight, ((0, 0), (0, GDIM - DIM)))
    g = _make_gather()
    idxs, quants = [], []
    for ci in range(NCHUNK):
        idx_c = _chunk_indices(flat, emb_weight, ci)
        idxs.append(idx_c)
        quants.append(g(emb_pad, idx_c))
    idx = jnp.concatenate(idxs)
    quantized = jnp.concatenate(quants)[:, :DIM]
    return (quantized.reshape(b, s, c), idx.reshape(b, s))
